# Initial kernel scaffold; baseline (speedup 1.0000x reference)
import jax
import jax.numpy as jnp
from jax.experimental import pallas as pl


def kernel(x, seg, tok_table, pos_table, seg_table, gamma, beta):
    raise NotImplementedError("write your pallas kernel here")


# speedup vs baseline: 3.9365x; 3.9365x over previous
import functools

import jax
import jax.numpy as jnp
from jax import lax
from jax.experimental import pallas as pl
from jax.experimental.pallas import tpu as pltpu
from jax.experimental.pallas import tpu_sc as plsc

D_MODEL = 768
MAXLEN = 30
N_SEGMENTS = 2
VOCAB = 4
N_COMBO = VOCAB * N_SEGMENTS * MAXLEN
_EPS = 1e-5

_NC = 2
_NS = 16
_NW = _NC * _NS
_LANES = 16
_G = 64


def _combo_body(tok_ref, pos_ref, seg_ref, gamma_ref, beta_ref, out_ref):
    row = lax.broadcasted_iota(jnp.int32, (N_COMBO, 1), 0)

    def onehot(ids, n):
        cols = lax.broadcasted_iota(jnp.int32, (N_COMBO, n), 1)
        return (ids == cols).astype(jnp.float32)

    emb = jnp.dot(onehot(row // (N_SEGMENTS * MAXLEN), VOCAB), tok_ref[...],
                  preferred_element_type=jnp.float32)
    emb = emb + jnp.dot(onehot((row // MAXLEN) % N_SEGMENTS, N_SEGMENTS),
                        seg_ref[...], preferred_element_type=jnp.float32)
    emb = emb + jnp.dot(onehot(row % MAXLEN, MAXLEN), pos_ref[...],
                        preferred_element_type=jnp.float32)
    mean = jnp.mean(emb, axis=-1, keepdims=True)
    cent = emb - mean
    var = jnp.mean(cent * cent, axis=-1, keepdims=True)
    out_ref[...] = (cent * lax.rsqrt(var + _EPS)) * gamma_ref[...] + beta_ref[...]


@functools.lru_cache(maxsize=None)
def _sc_lookup(ntok: int, seq_len: int):
    assert ntok % (_NW * _G) == 0
    b_w = ntok // _NW
    n_vec = b_w // _LANES
    n_chunk = b_w // _G
    assert n_chunk >= 2 and n_chunk % 2 == 0
    mesh = plsc.VectorSubcoreMesh(core_axis_name="c", subcore_axis_name="s")

    @functools.partial(
        pl.kernel,
        mesh=mesh,
        out_type=jax.ShapeDtypeStruct((ntok, D_MODEL), jnp.float32),
        scratch_types=[
            pltpu.VMEM((b_w,), jnp.int32),
            pltpu.VMEM((b_w,), jnp.int32),
            pltpu.VMEM((b_w,), jnp.int32),
            pltpu.VMEM((2, _G, D_MODEL), jnp.float32),
            pltpu.SemaphoreType.DMA,
            pltpu.SemaphoreType.DMA,
        ],
    )
    def body(x_hbm, s_hbm, combo_hbm, out_hbm, x_v, s_v, idx_v, buf_v, sem0, sem1):
        sems = (sem0, sem1)
        wid = lax.axis_index("s") * _NC + lax.axis_index("c")
        base = pl.multiple_of(wid * b_w, b_w)
        pltpu.sync_copy(x_hbm.at[pl.ds(base, b_w)], x_v)
        pltpu.sync_copy(s_hbm.at[pl.ds(base, b_w)], s_v)

        lanes = lax.iota(jnp.int32, _LANES)

        def cid_body(j, carry):
            off = pl.multiple_of(j * _LANES, _LANES)
            xv = x_v[pl.ds(off, _LANES)]
            sv = s_v[pl.ds(off, _LANES)]
            t = (base + off + lanes) % seq_len
            idx_v[pl.ds(off, _LANES)] = (xv * N_SEGMENTS + sv) * seq_len + t
            return carry

        lax.fori_loop(0, n_vec, cid_body, 0)

        def gather_chunk(k, b):
            off = pl.multiple_of(k * _G, _G)
            return pltpu.async_copy(
                combo_hbm.at[idx_v.at[pl.ds(off, _G)]], buf_v.at[b], sems[b])

        gather_chunk(0, 0)
        gather_chunk(1, 1)

        def outer(i, carry):
            for b in range(2):
                k = i * 2 + b
                off = pl.multiple_of(k * _G, _G)
                pltpu.make_async_copy(
                    combo_hbm.at[idx_v.at[pl.ds(off, _G)]], buf_v.at[b], sems[b]
                ).wait()
                pltpu.sync_copy(buf_v.at[b], out_hbm.at[pl.ds(base + off, _G)])

                @pl.when(k + 2 < n_chunk)
                def _start_next():
                    gather_chunk(k + 2, b)

            return carry

        lax.fori_loop(0, n_chunk // 2, outer, 0)

    return body


def kernel(x, seg, tok_table, pos_table, seg_table, gamma, beta):
    combo = pl.pallas_call(
        _combo_body,
        out_shape=jax.ShapeDtypeStruct((N_COMBO, D_MODEL), jnp.float32),
    )(tok_table, pos_table, seg_table,
      gamma.reshape(1, D_MODEL).astype(jnp.float32),
      beta.reshape(1, D_MODEL).astype(jnp.float32))

    batch, seq_len = x.shape
    ntok = batch * seq_len
    xf = x.reshape(ntok).astype(jnp.int32)
    sf = seg.reshape(ntok).astype(jnp.int32)
    out = _sc_lookup(ntok, seq_len)(xf, sf, combo)
    return out.reshape(batch, seq_len, D_MODEL)
